# baseline (speedup 1.0000x reference)
import functools

import jax
import jax.numpy as jnp
from jax import lax
from jax.experimental import pallas as pl
from jax.experimental.pallas import tpu as pltpu
from jax.experimental.pallas import tpu_sc as plsc

NP = 10240
NT = NP // 32
CH = 256


def _edge_dot(hs_bf16, ewp_bf16, w_f32):
    E = hs_bf16.shape[0]
    BLK = 1280

    def body(h_ref, e_ref, w_ref, o_ref):
        m2 = jnp.concatenate([h_ref[...], e_ref[...]], axis=1)
        acc = lax.dot_general(m2, w_ref[...],
                              dimension_numbers=(((1,), (0,)), ((), ())),
                              preferred_element_type=jnp.float32)
        o_ref[...] = jnp.where(acc >= 0, acc, acc * jnp.float32(0.01))

    return pl.pallas_call(
        body,
        grid=(E // BLK,),
        in_specs=[pl.BlockSpec((BLK, 128), lambda i: (i, 0)),
                  pl.BlockSpec((BLK, 128), lambda i: (i, 0)),
                  pl.BlockSpec((256, 128), lambda i: (0, 0))],
        out_specs=pl.BlockSpec((BLK, 128), lambda i: (i, 0)),
        out_shape=jax.ShapeDtypeStruct((E + CH, 128), jnp.float32),
    )(hs_bf16, ewp_bf16, w_f32)


NZW = 352


def _sc_accum(t_pad, nzdeg, nzdst, meta):
    mesh = plsc.VectorSubcoreMesh(core_axis_name="c", subcore_axis_name="s")

    @functools.partial(
        pl.kernel, mesh=mesh,
        out_type=jax.ShapeDtypeStruct((NP, 128), jnp.float32),
        scratch_types=[
            pltpu.VMEM((CH, 128), jnp.float32),
            pltpu.VMEM((NT + 8, 128), jnp.float32),
            pltpu.VMEM((NZW * 16 + 16,), jnp.int32),
            pltpu.VMEM((NZW * 16 + 16,), jnp.int32),
            pltpu.VMEM((5 * 32 * 16,), jnp.int32),
        ],
    )
    def k(t_hbm, nzdeg_hbm, nzdst_hbm, meta_hbm, out_hbm,
          tbuf, out_v, nzdeg_v, nzdst_v, meta_v):
        c = lax.axis_index("c")
        s_ = lax.axis_index("s")
        wid = s_ * 2 + c
        lo = wid * NT

        def sload(ref, idx):
            b = pl.multiple_of(idx * 16, 8)
            return ref[pl.ds(b, 16)][0]

        pltpu.sync_copy(meta_hbm, meta_v)
        pltpu.sync_copy(nzdeg_hbm.at[pl.ds(wid * NZW * 16, NZW * 16)],
                        nzdeg_v.at[pl.ds(0, NZW * 16)])
        pltpu.sync_copy(nzdst_hbm.at[pl.ds(wid * NZW * 16, NZW * 16)],
                        nzdst_v.at[pl.ds(0, NZW * 16)])
        nzdeg_v[pl.ds(NZW * 16, 16)] = jnp.full((16,), 1 << 30, jnp.int32)
        nzdst_v[pl.ds(NZW * 16, 16)] = jnp.full((16,), NT, jnp.int32)

        abase = sload(meta_v, wid)
        nch = sload(meta_v, 32 + wid)
        j0 = sload(meta_v, 64 + wid)
        rem0 = sload(meta_v, 96 + wid)
        cur0 = sload(meta_v, 128 + wid)

        zero = jnp.zeros((16,), jnp.float32)

        def zloop(i, _):
            for j in range(8):
                out_v[i, pl.ds(16 * j, 16)] = zero
            return 0
        lax.fori_loop(0, NT, zloop, 0)

        def row_body(r, carry2):
            jj, cur, rem, acc = carry2
            newacc = tuple(
                acc[q] + tbuf[r, pl.ds(16 * q, 16)] for q in range(8))
            for q in range(8):
                out_v[cur, pl.ds(16 * q, 16)] = newacc[q]
            done = rem == 1
            j2 = jnp.where(done, jj + 1, jj)
            rem2 = jnp.where(done, sload(nzdeg_v, j2), rem - 1)
            cur2 = jnp.where(done, sload(nzdst_v, j2), cur)
            acc2 = tuple(
                jnp.where(done, zero, newacc[q]) for q in range(8))
            return j2, cur2, rem2, acc2

        def chunk_body(kk, carry):
            jj, cur, rem, acc = carry
            base = abase + kk * CH
            base = pl.multiple_of(base, 8)
            pltpu.sync_copy(t_hbm.at[pl.ds(base, CH)], tbuf)
            return lax.fori_loop(0, CH, row_body, (jj, cur, rem, acc))

        acc0 = (zero,) * 8
        lax.fori_loop(0, nch, chunk_body, (j0, cur0, rem0, acc0))
        pltpu.sync_copy(out_v.at[pl.ds(0, NT)], out_hbm.at[pl.ds(lo, NT)])

    return k(t_pad, nzdeg, nzdst, meta)


def _topk_select(key_col, key_row, h_pad):
    BR = 1024
    G = NP // BR

    def body(kc_ref, kr_ref, h_ref, sel_ref, idx_ref):
        i = pl.program_id(0)
        ke = kc_ref[...]
        kj = kr_ref[...]
        jj = jax.lax.broadcasted_iota(jnp.int32, (1, NP), 1)
        ee = (jax.lax.broadcasted_iota(jnp.int32, (BR, 1), 0) + i * BR)
        gt = (kj > ke).astype(jnp.int32)
        tie = jnp.logical_and(kj == ke, jj < ee).astype(jnp.int32)
        rank = jnp.sum(gt + tie, axis=1, keepdims=True)
        rr = jax.lax.broadcasted_iota(jnp.int32, (1, 512), 1)
        oh = jnp.logical_and(rank == rr, rank < 512)
        ohf = oh.astype(jnp.float32)
        sel_p = jax.lax.dot_general(
            ohf, h_ref[...], (((0,), (0,)), ((), ())),
            preferred_element_type=jnp.float32)
        col = jax.lax.broadcasted_iota(jnp.int32, (BR, 128), 1)
        ef = ee.astype(jnp.float32)
        idx_rhs = jnp.where(col == 0, jnp.floor_divide(ee, 128)
                            .astype(jnp.float32),
                            jnp.where(col == 1,
                                      (ee % 128).astype(jnp.float32), 0.0))
        idx_p = jax.lax.dot_general(
            ohf, idx_rhs, (((0,), (0,)), ((), ())),
            preferred_element_type=jnp.float32)

        @pl.when(i == 0)
        def _():
            sel_ref[...] = jnp.zeros_like(sel_ref)
            idx_ref[...] = jnp.zeros_like(idx_ref)
        sel_ref[...] += sel_p
        idx_ref[...] += idx_p

    return pl.pallas_call(
        body,
        grid=(G,),
        in_specs=[pl.BlockSpec((BR, 1), lambda i: (i, 0)),
                  pl.BlockSpec((1, NP), lambda i: (0, 0)),
                  pl.BlockSpec((BR, 128), lambda i: (i, 0))],
        out_specs=[pl.BlockSpec((512, 128), lambda i: (0, 0)),
                   pl.BlockSpec((512, 128), lambda i: (0, 0))],
        out_shape=[jax.ShapeDtypeStruct((512, 128), jnp.float32),
                   jax.ShapeDtypeStruct((512, 128), jnp.float32)],
    )(key_col, key_row, h_pad)


def _actor(sel, aw1, ab1, aw2, ab2, masks_f):
    def body(s_ref, w1_ref, b1_ref, w2_ref, b2_ref, m_ref, o_ref):
        a1 = jax.lax.dot_general(
            s_ref[...].astype(jnp.bfloat16), w1_ref[...],
            (((1,), (0,)), ((), ())),
            preferred_element_type=jnp.float32)
        a1 = jnp.maximum(a1 + b1_ref[...], 0.0)
        lg = jax.lax.dot_general(
            a1.astype(jnp.bfloat16), w2_ref[...],
            (((1,), (0,)), ((), ())),
            preferred_element_type=jnp.float32)
        lg = lg + b2_ref[...]
        lg = jnp.where(m_ref[...] > 0.5, lg, jnp.float32(-1e9))
        mx = jnp.max(lg, axis=1, keepdims=True)
        ex = jnp.exp(lg - mx)
        o_ref[...] = ex / jnp.sum(ex, axis=1, keepdims=True)

    return pl.pallas_call(
        body,
        in_specs=[pl.BlockSpec(sel.shape, lambda: (0, 0)),
                  pl.BlockSpec(aw1.shape, lambda: (0, 0)),
                  pl.BlockSpec(ab1.shape, lambda: (0, 0)),
                  pl.BlockSpec(aw2.shape, lambda: (0, 0)),
                  pl.BlockSpec(ab2.shape, lambda: (0, 0)),
                  pl.BlockSpec(masks_f.shape, lambda: (0, 0))],
        out_specs=pl.BlockSpec((512, 512), lambda: (0, 0)),
        out_shape=jax.ShapeDtypeStruct((512, 512), jnp.float32),
    )(sel, aw1, ab1, aw2, ab2, masks_f)


def kernel(gate_type, edge_index, edge_w, masks, k, emb, c0_w1, c0_w2, c0_b2,
           cs_w1, cs_w2, cs_b2, actor_w1, actor_b1, actor_w2, actor_b2,
           critic_w1, critic_b1, critic_w2, critic_b2):
    n = gate_type.shape[0]
    D = 128
    NGT = emb.shape[0]
    E = edge_index.shape[1]
    src = edge_index[0]
    dst = edge_index[1]
    bf = jnp.bfloat16

    perm = jnp.argsort(dst, stable=True)
    dst_sorted = dst[perm]
    start = jnp.searchsorted(dst_sorted, jnp.arange(NP + 1),
                             side="left").astype(jnp.int32)
    deg_i = start[1:] - start[:-1]
    deg_f = deg_i[:n].astype(jnp.float32)
    src_s = src[perm]
    ew_s = edge_w[perm]

    i32 = jnp.int32
    deg2 = deg_i.reshape(32, NT)
    ar = jnp.arange(NT, dtype=i32)
    key = jnp.where(deg2 > 0, ar[None, :], 1 << 20)
    order = jnp.argsort(key, axis=1).astype(i32)
    sdeg = jnp.take_along_axis(deg2, order, axis=1)
    nzdeg_real = jnp.where(sdeg > 0, sdeg, 1 << 30)
    nzdst_real = jnp.where(sdeg > 0, order, NT)
    start_w = start[:-1:NT]
    end_w = start[NT::NT]
    aoff = start_w % 8
    abase = start_w - aoff
    Lw = end_w - start_w
    nch = (aoff + Lw + CH - 1) // CH
    pad_w = NZW - 1 - NT
    nzdeg2 = jnp.concatenate(
        [aoff[:, None], nzdeg_real,
         jnp.full((32, pad_w), 1 << 30, i32)], axis=1)
    nzdst2 = jnp.concatenate(
        [jnp.full((32, 1), NT, i32), nzdst_real,
         jnp.full((32, pad_w), NT, i32)], axis=1)
    initj = (aoff == 0).astype(i32)
    initrem = jnp.take_along_axis(nzdeg2, initj[:, None], axis=1)[:, 0]
    initcur = jnp.take_along_axis(nzdst2, initj[:, None], axis=1)[:, 0]
    meta0 = jnp.concatenate([abase, nch, initj, initrem, initcur]).astype(i32)
    nzdeg = jnp.repeat(nzdeg2.reshape(-1), 16)
    nzdst = jnp.repeat(nzdst2.reshape(-1), 16)
    meta = jnp.repeat(meta0, 16)
    ewp = jnp.pad(ew_s.astype(bf), ((0, 0), (0, 125)))

    h = emb[gate_type]
    h_bf = jnp.pad(h.astype(bf), ((0, 0), (0, D - NGT)))

    for i in range(6):
        if i == 0:
            w1h = jnp.pad(c0_w1[:NGT], ((0, D - NGT), (0, 0)))
            w1e = c0_w1[NGT:]
            w2h = jnp.pad(c0_w2[:NGT], ((0, D - NGT), (0, 0)))
            w2b = c0_w2[NGT:]
            b2 = c0_b2
        else:
            w1h = cs_w1[i - 1][:D]
            w1e = cs_w1[i - 1][D:]
            w2h = cs_w2[i - 1][:D]
            w2b = cs_w2[i - 1][D:]
            b2 = cs_b2[i - 1]
        w1_p = jnp.concatenate(
            [w1h, w1e, jnp.zeros((125, D), jnp.float32)], axis=0)
        hs = h_bf[src_s]
        t = _edge_dot(hs, ewp, w1_p)
        s = jax.ops.segment_sum(t[:E], dst_sorted, num_segments=NP)[:n]
        h_N = s / jnp.maximum(deg_f, 1.0)[:, None]
        if i == 0:
            h_total = jnp.concatenate([h, h_N], axis=1)
            w2_full = jnp.concatenate([c0_w2[:NGT], w2b], axis=0)
            h = jax.nn.relu(h_total @ w2_full + b2)
        else:
            h_total = jnp.concatenate([h, h_N], axis=1)
            h = jax.nn.relu(h_total @ jnp.concatenate([w2h, w2b], axis=0)
                            + b2)
        h_bf = h.astype(bf)

    node_values = (jax.nn.relu(h @ critic_w1 + critic_b1) @ critic_w2
                   + critic_b2).squeeze(-1)

    v0 = node_values + jnp.float32(0.0)
    b = jax.lax.bitcast_convert_type(v0, jnp.int32)
    keyv = b ^ ((b >> 31) & jnp.int32(0x7FFFFFFF))
    keyp = jnp.full((NP,), jnp.iinfo(jnp.int32).min, jnp.int32
                    ).at[:n].set(keyv)
    h_pad = jnp.pad(h, ((0, NP - n), (0, 0)))
    sel, idxp = _topk_select(keyp[:, None], keyp[None, :], h_pad)
    node_idxs = (jnp.round(idxp[:, 0]).astype(jnp.int32) * 128
                 + jnp.round(idxp[:, 1]).astype(jnp.int32))
    xfer_probs = _actor(sel, actor_w1, actor_b1[None, :], actor_w2,
                        actor_b2[None, :], masks.astype(jnp.float32))
    return xfer_probs, node_idxs
